# baseline (speedup 1.0000x reference)
import jax
import jax.numpy as jnp
from jax.experimental import pallas as pl

_N = 64
_MIN_DEPTH = 2.0
_MAX_DEPTH = 6.0
_BLK = 2048


def _raysample_kernel(o_ref, d_ref, pts_ref, len_ref):
    blk = o_ref.shape[0]
    step = (_MAX_DEPTH - _MIN_DEPTH) / (_N - 1)
    k = jax.lax.broadcasted_iota(jnp.int32, (1, _N * 3), 1)
    zf = _MIN_DEPTH + (k // 3).astype(jnp.float32) * step
    m = k % 3
    o = o_ref[...]
    d = d_ref[...]
    acc = jnp.zeros((blk, _N * 3), jnp.float32)
    for c in range(3):
        val = o[:, c : c + 1] + d[:, c : c + 1] * zf
        acc = jnp.where(m == c, val, acc)
    pts_ref[...] = acc
    kn = jax.lax.broadcasted_iota(jnp.int32, (1, _N), 1).astype(jnp.float32)
    len_ref[...] = jnp.broadcast_to(_MIN_DEPTH + kn * step, (blk, _N))


@jax.jit
def kernel(origins, directions):
    B = origins.shape[0]
    pts2d, len2d = pl.pallas_call(
        _raysample_kernel,
        grid=(B // _BLK,),
        in_specs=[
            pl.BlockSpec((_BLK, 3), lambda i: (i, 0)),
            pl.BlockSpec((_BLK, 3), lambda i: (i, 0)),
        ],
        out_specs=[
            pl.BlockSpec((_BLK, _N * 3), lambda i: (i, 0)),
            pl.BlockSpec((_BLK, _N), lambda i: (i, 0)),
        ],
        out_shape=[
            jax.ShapeDtypeStruct((B, _N * 3), jnp.float32),
            jax.ShapeDtypeStruct((B, _N), jnp.float32),
        ],
    )(origins, directions)
    return pts2d.reshape(B, _N, 3), len2d.reshape(B, _N, 1)


# speedup vs baseline: 9.4539x; 9.4539x over previous
import jax
import jax.numpy as jnp
from jax.experimental import pallas as pl

_N = 64
_MIN_DEPTH = 2.0
_MAX_DEPTH = 6.0
_BLKB = 2048
_LANES = 128


def _raysample_kernel(o_ref, d_ref, pts_ref, len_ref):
    step = (_MAX_DEPTH - _MIN_DEPTH) / (_N - 1)
    z = _MIN_DEPTH + step * jax.lax.broadcasted_iota(
        jnp.int32, (1, _N, 1), 1
    ).astype(jnp.float32)
    o = o_ref[...]
    d = d_ref[...]
    pts_ref[...] = o[:, None, :] + d[:, None, :] * z
    zl = _MIN_DEPTH + step * jax.lax.broadcasted_iota(
        jnp.int32, (_N, 1, 1), 0
    ).astype(jnp.float32)
    len_ref[...] = jnp.broadcast_to(zl, len_ref.shape)


@jax.jit
def kernel(origins, directions):
    B = origins.shape[0]
    o_t = origins.T
    d_t = directions.T
    pts_t, len_t = pl.pallas_call(
        _raysample_kernel,
        grid=(B // _BLKB,),
        in_specs=[
            pl.BlockSpec((3, _BLKB), lambda i: (0, i)),
            pl.BlockSpec((3, _BLKB), lambda i: (0, i)),
        ],
        out_specs=[
            pl.BlockSpec((3, _N, _BLKB), lambda i: (0, 0, i)),
            pl.BlockSpec((_N, _BLKB // _LANES, _LANES), lambda i: (0, i, 0)),
        ],
        out_shape=[
            jax.ShapeDtypeStruct((3, _N, B), jnp.float32),
            jax.ShapeDtypeStruct((_N, B // _LANES, _LANES), jnp.float32),
        ],
    )(o_t, d_t)
    pts = jnp.transpose(pts_t, (2, 1, 0))
    lengths = jax.lax.reshape(len_t, (B, _N, 1), dimensions=(1, 2, 0))
    return pts, lengths


# speedup vs baseline: 12.9413x; 1.3689x over previous
import jax
import jax.numpy as jnp
from jax.experimental import pallas as pl

_N = 64
_MIN_DEPTH = 2.0
_MAX_DEPTH = 6.0
_BLKB = 8192
_LANES = 128


def _raysample_kernel(o_ref, d_ref, pts_ref, len_ref):
    step = (_MAX_DEPTH - _MIN_DEPTH) / (_N - 1)
    z = _MIN_DEPTH + step * jax.lax.broadcasted_iota(
        jnp.int32, (1, _N, 1), 1
    ).astype(jnp.float32)
    o = o_ref[...]
    d = d_ref[...]
    pts_ref[...] = o[:, None, :] + d[:, None, :] * z
    zl = _MIN_DEPTH + step * jax.lax.broadcasted_iota(
        jnp.int32, (_N, 1, 1), 0
    ).astype(jnp.float32)
    len_ref[...] = jnp.broadcast_to(zl, len_ref.shape)


@jax.jit
def kernel(origins, directions):
    B = origins.shape[0]
    o_t = origins.T
    d_t = directions.T
    pts_t, len_t = pl.pallas_call(
        _raysample_kernel,
        grid=(B // _BLKB,),
        in_specs=[
            pl.BlockSpec((3, _BLKB), lambda i: (0, i)),
            pl.BlockSpec((3, _BLKB), lambda i: (0, i)),
        ],
        out_specs=[
            pl.BlockSpec((3, _N, _BLKB), lambda i: (0, 0, i)),
            pl.BlockSpec((_N, _BLKB // _LANES, _LANES), lambda i: (0, i, 0)),
        ],
        out_shape=[
            jax.ShapeDtypeStruct((3, _N, B), jnp.float32),
            jax.ShapeDtypeStruct((_N, B // _LANES, _LANES), jnp.float32),
        ],
    )(o_t, d_t)
    pts = jnp.transpose(pts_t, (2, 1, 0))
    lengths = jax.lax.reshape(len_t, (B, _N, 1), dimensions=(1, 2, 0))
    return pts, lengths
